# Initial kernel scaffold; baseline (speedup 1.0000x reference)
import jax
import jax.numpy as jnp
from jax.experimental import pallas as pl


def kernel(x, edge_index, W0, W1):
    raise NotImplementedError("write your pallas kernel here")


# speedup vs baseline: 18.0850x; 18.0850x over previous
import functools

import jax
import jax.numpy as jnp
from jax import lax
from jax.experimental import pallas as pl
from jax.experimental.pallas import tpu as pltpu
from jax.experimental.pallas import tpu_sc as plsc

N = 10000
E = 320000
NPAD = 10112
CHUNK = 128
NW = 32
CH_PER_W = 79
EP = CHUNK * NW * CH_PER_W
RPT = NPAD // 16


def _fill_vmem_2d(ref, nrows, ncols, value):
    v = jnp.full((16,), value, jnp.float32)

    def body(r, _):
        for j in range(ncols // 16):
            ref[r, pl.ds(j * 16, 16)] = v
        return 0

    lax.fori_loop(0, nrows, body, 0)


def _copy_rows(src_ref, dst_ref, r0):
    for p in range(RPT // CHUNK):
        pltpu.sync_copy(src_ref, dst_ref.at[pl.ds(r0 + p * CHUNK, CHUNK), :])
    rem = RPT % CHUNK
    if rem:
        pltpu.sync_copy(
            src_ref.at[pl.ds(0, rem), :],
            dst_ref.at[pl.ds(r0 + (RPT // CHUNK) * CHUNK, rem), :],
        )


def _sc_degree(dst_pad):
    mesh = plsc.VectorSubcoreMesh(core_axis_name="c", subcore_axis_name="s")

    @functools.partial(
        pl.kernel,
        out_type=jax.ShapeDtypeStruct((2, NPAD, 16), jnp.float32),
        mesh=mesh,
        scratch_types=[
            pltpu.VMEM((CHUNK,), jnp.int32),
            pltpu.VMEM((CHUNK, 16), jnp.float32),
            pltpu.VMEM_SHARED((NPAD, 16), jnp.float32),
        ],
    )
    def k(dst_hbm, out_hbm, didx, buf, acc):
        c = lax.axis_index("c")
        s = lax.axis_index("s")
        wid = c * 16 + s
        r0 = s * RPT

        _fill_vmem_2d(buf, CHUNK, 16, 0.0)
        _copy_rows(buf, acc, r0)
        _fill_vmem_2d(buf, CHUNK, 16, 1.0)
        plsc.subcore_barrier()

        def body(j, _):
            base = (wid * CH_PER_W + j) * CHUNK
            pltpu.sync_copy(dst_hbm.at[pl.ds(base, CHUNK)], didx)
            pltpu.sync_copy(buf, acc.at[didx], add=True)
            return 0

        lax.fori_loop(0, CH_PER_W, body, 0)
        plsc.subcore_barrier()
        pltpu.sync_copy(acc.at[pl.ds(r0, RPT), :],
                        out_hbm.at[c, pl.ds(r0, RPT), :])

    return k(dst_pad)


def _sc_propagate(table, src_pad, dst_pad, d):
    mesh = plsc.VectorSubcoreMesh(core_axis_name="c", subcore_axis_name="s")

    @functools.partial(
        pl.kernel,
        out_type=jax.ShapeDtypeStruct((2, NPAD, d), jnp.float32),
        mesh=mesh,
        compiler_params=pltpu.CompilerParams(use_tc_tiling_on_sc=False),
        scratch_types=[
            pltpu.VMEM((CHUNK,), jnp.int32),
            pltpu.VMEM((CHUNK,), jnp.int32),
            pltpu.VMEM((CHUNK, d), jnp.float32),
            pltpu.VMEM_SHARED((NPAD, d), jnp.float32),
            pltpu.SemaphoreType.DMA,
        ],
    )
    def k(tab_hbm, src_hbm, dst_hbm, out_hbm, sidx, didx, rows, acc, sem):
        c = lax.axis_index("c")
        s = lax.axis_index("s")
        wid = c * 16 + s
        r0 = s * RPT

        _fill_vmem_2d(rows, CHUNK, d, 0.0)
        _copy_rows(rows, acc, r0)
        plsc.subcore_barrier()

        def body(j, _):
            base = (wid * CH_PER_W + j) * CHUNK
            pltpu.sync_copy(src_hbm.at[pl.ds(base, CHUNK)], sidx)
            pltpu.sync_copy(dst_hbm.at[pl.ds(base, CHUNK)], didx)
            pltpu.async_copy(tab_hbm.at[sidx], rows, sem).wait()
            pltpu.sync_copy(rows, acc.at[didx], add=True)
            return 0

        lax.fori_loop(0, CH_PER_W, body, 0)
        plsc.subcore_barrier()
        pltpu.sync_copy(acc.at[pl.ds(r0, RPT), :],
                        out_hbm.at[c, pl.ds(r0, RPT), :])

    return k(table, src_pad, dst_pad)


def _tc_layer1(x, w0, degp):
    def body(x_ref, w0_ref, degp_ref, hs_ref, dinv_ref):
        deg = degp_ref[0, :N, 0:1] + degp_ref[1, :N, 0:1] + 1.0
        dinv = lax.rsqrt(deg)
        dinv_b = jnp.broadcast_to(dinv, (N, 128))
        dinv_ref[...] = dinv_b
        h = jnp.dot(x_ref[...], w0_ref[...], preferred_element_type=jnp.float32)
        hs_ref[...] = h * dinv_b

    return pl.pallas_call(
        body,
        out_shape=(
            jax.ShapeDtypeStruct((N, 128), jnp.float32),
            jax.ShapeDtypeStruct((N, 128), jnp.float32),
        ),
    )(x, w0, degp)


def _tc_layer2(part1, hs, dinv_b, w1):
    def body(p_ref, hs_ref, dinv_ref, w1_ref, out_ref):
        acc = p_ref[0, :N, :] + p_ref[1, :N, :] + hs_ref[...]
        h1 = jnp.maximum(dinv_ref[...] * acc, 0.0)
        h2 = jnp.dot(h1, w1_ref[...], preferred_element_type=jnp.float32)
        out_ref[...] = h2 * dinv_ref[:, :64]

    return pl.pallas_call(
        body,
        out_shape=jax.ShapeDtypeStruct((N, 64), jnp.float32),
    )(part1, hs, dinv_b, w1)


def _tc_final(part2, hs2, dinv_b):
    def body(p_ref, hs2_ref, dinv_ref, out_ref):
        acc = p_ref[0, :N, :] + p_ref[1, :N, :] + hs2_ref[...]
        out_ref[...] = dinv_ref[:, :64] * acc

    return pl.pallas_call(
        body,
        out_shape=jax.ShapeDtypeStruct((N, 64), jnp.float32),
    )(part2, hs2, dinv_b)


def kernel(x, edge_index, W0, W1):
    src = edge_index[0].astype(jnp.int32)
    dst = edge_index[1].astype(jnp.int32)
    npad = EP - E
    pad = jnp.arange(npad, dtype=jnp.int32)
    src_pad = jnp.concatenate([src, pad % N])
    dst_pad = jnp.concatenate([dst, N + pad % (NPAD - N)])

    degp = _sc_degree(dst_pad)
    hs, dinv_b = _tc_layer1(x, W0, degp)
    part1 = _sc_propagate(hs, src_pad, dst_pad, 128)
    hs2 = _tc_layer2(part1, hs, dinv_b, W1)
    part2 = _sc_propagate(hs2, src_pad, dst_pad, 64)
    return _tc_final(part2, hs2, dinv_b)


# speedup vs baseline: 39.0232x; 2.1578x over previous
import functools

import jax
import jax.numpy as jnp
from jax import lax
from jax.experimental import pallas as pl
from jax.experimental.pallas import tpu as pltpu
from jax.experimental.pallas import tpu_sc as plsc

N = 10000
E = 320000
NPAD = 10112
CHUNK = 128
NW = 32
CH_PER_W = 80
EP = CHUNK * NW * CH_PER_W
RPT = NPAD // 16


def _fill_vmem_2d(ref, nrows, ncols, value):
    v = jnp.full((16,), value, jnp.float32)

    def body(r, _):
        for j in range(ncols // 16):
            ref[r, pl.ds(j * 16, 16)] = v
        return 0

    lax.fori_loop(0, nrows, body, 0)


def _copy_rows(src_ref, dst_ref, r0):
    for p in range(RPT // CHUNK):
        pltpu.sync_copy(src_ref, dst_ref.at[pl.ds(r0 + p * CHUNK, CHUNK), :])
    rem = RPT % CHUNK
    if rem:
        pltpu.sync_copy(
            src_ref.at[pl.ds(0, rem), :],
            dst_ref.at[pl.ds(r0 + (RPT // CHUNK) * CHUNK, rem), :],
        )


def _sc_degree(dst3):
    mesh = plsc.VectorSubcoreMesh(core_axis_name="c", subcore_axis_name="s", num_cores=2, num_subcores=16)
    nbuf = 4

    @functools.partial(
        pl.kernel,
        out_type=jax.ShapeDtypeStruct((2, NPAD, 16), jnp.float32),
        mesh=mesh,
        compiler_params=pltpu.CompilerParams(use_tc_tiling_on_sc=False),
        scratch_types=(
            [pltpu.VMEM((CHUNK,), jnp.int32)] * nbuf
            + [pltpu.VMEM((CHUNK, 16), jnp.float32)]
            + [pltpu.VMEM_SHARED((NPAD, 16), jnp.float32)]
            + [pltpu.SemaphoreType.DMA] * nbuf
        ),
    )
    def k(dst_hbm, out_hbm, *scr):
        didx = scr[:nbuf]
        buf = scr[nbuf]
        acc = scr[nbuf + 1]
        dsem = scr[nbuf + 2:]
        c = lax.axis_index("c")
        s = lax.axis_index("s")
        wid = c * 16 + s
        r0 = s * RPT

        _fill_vmem_2d(buf, CHUNK, 16, 0.0)
        _copy_rows(buf, acc, r0)
        _fill_vmem_2d(buf, CHUNK, 16, 1.0)
        plsc.subcore_barrier()

        for b in range(nbuf):
            pltpu.async_copy(dst_hbm.at[wid, b], didx[b], dsem[b])

        def body(g, _):
            for b in range(nbuf):
                j = g * nbuf + b
                pltpu.make_async_copy(dst_hbm.at[wid, 0], didx[b],
                                      dsem[b]).wait()
                pltpu.sync_copy(buf, acc.at[didx[b]], add=True)

                @pl.when(j + nbuf < CH_PER_W)
                def _issue_next():
                    pltpu.async_copy(dst_hbm.at[wid, j + nbuf], didx[b],
                                     dsem[b])
            return 0

        lax.fori_loop(0, CH_PER_W // nbuf, body, 0)
        plsc.subcore_barrier()
        pltpu.sync_copy(acc.at[pl.ds(r0, RPT), :],
                        out_hbm.at[c, pl.ds(r0, RPT), :])

    return k(dst3)


def _sc_propagate(table, src3, dst3, d, nbuf):
    mesh = plsc.VectorSubcoreMesh(core_axis_name="c", subcore_axis_name="s", num_cores=2, num_subcores=16)
    assert CH_PER_W % nbuf == 0

    @functools.partial(
        pl.kernel,
        out_type=jax.ShapeDtypeStruct((2, NPAD, d), jnp.float32),
        mesh=mesh,
        compiler_params=pltpu.CompilerParams(use_tc_tiling_on_sc=False),
        scratch_types=(
            [pltpu.VMEM((CH_PER_W, CHUNK), jnp.int32)]
            + [pltpu.VMEM((CHUNK,), jnp.int32)] * nbuf
            + [pltpu.VMEM((CHUNK, d), jnp.float32)] * nbuf
            + [pltpu.VMEM_SHARED((NPAD, d), jnp.float32)]
            + [pltpu.SemaphoreType.DMA] * (2 * nbuf)
        ),
    )
    def k(tab_hbm, src_hbm, dst_hbm, out_hbm, sidx, *scr):
        didx = scr[:nbuf]
        rows = scr[nbuf:2 * nbuf]
        acc = scr[2 * nbuf]
        dsem = scr[2 * nbuf + 1:3 * nbuf + 1]
        gsem = scr[3 * nbuf + 1:]
        c = lax.axis_index("c")
        s = lax.axis_index("s")
        wid = c * 16 + s
        r0 = s * RPT

        pltpu.sync_copy(src_hbm.at[wid], sidx)
        _fill_vmem_2d(rows[0], CHUNK, d, 0.0)
        _copy_rows(rows[0], acc, r0)
        plsc.subcore_barrier()

        for b in range(nbuf):
            pltpu.async_copy(dst_hbm.at[wid, b], didx[b], dsem[b])
            pltpu.async_copy(tab_hbm.at[sidx.at[b]], rows[b], gsem[b])

        def body(g, _):
            for b in range(nbuf):
                j = g * nbuf + b
                pltpu.make_async_copy(dst_hbm.at[wid, 0], didx[b],
                                      dsem[b]).wait()
                pltpu.make_async_copy(tab_hbm.at[sidx.at[0]], rows[b],
                                      gsem[b]).wait()
                pltpu.sync_copy(rows[b], acc.at[didx[b]], add=True)

                @pl.when(j + nbuf < CH_PER_W)
                def _issue_next():
                    pltpu.async_copy(dst_hbm.at[wid, j + nbuf], didx[b],
                                     dsem[b])
                    pltpu.async_copy(tab_hbm.at[sidx.at[j + nbuf]],
                                     rows[b], gsem[b])
            return 0

        lax.fori_loop(0, CH_PER_W // nbuf, body, 0)
        plsc.subcore_barrier()
        pltpu.sync_copy(acc.at[pl.ds(r0, RPT), :],
                        out_hbm.at[c, pl.ds(r0, RPT), :])

    return k(table, src3, dst3)


def _tc_layer1(x, w0, degp):
    def body(x_ref, w0_ref, degp_ref, hs_ref, dinv_ref):
        deg = degp_ref[0, :N, 0:1] + degp_ref[1, :N, 0:1] + 1.0
        dinv = lax.rsqrt(deg)
        dinv_b = jnp.broadcast_to(dinv, (N, 128))
        dinv_ref[...] = dinv_b
        h = jnp.dot(x_ref[...], w0_ref[...],
                    preferred_element_type=jnp.float32)
        hs_ref[...] = h * dinv_b

    return pl.pallas_call(
        body,
        out_shape=(
            jax.ShapeDtypeStruct((N, 128), jnp.float32),
            jax.ShapeDtypeStruct((N, 128), jnp.float32),
        ),
    )(x, w0, degp)


def _tc_layer2(part1, hs, dinv_b, w1):
    def body(p_ref, hs_ref, dinv_ref, w1_ref, out_ref):
        acc = p_ref[0, :N, :] + p_ref[1, :N, :] + hs_ref[...]
        h1 = jnp.maximum(dinv_ref[...] * acc, 0.0)
        h2 = jnp.dot(h1, w1_ref[...], preferred_element_type=jnp.float32)
        out_ref[...] = h2 * dinv_ref[:, :64]

    return pl.pallas_call(
        body,
        out_shape=jax.ShapeDtypeStruct((N, 64), jnp.float32),
    )(part1, hs, dinv_b, w1)


def _tc_final(part2, hs2, dinv_b):
    def body(p_ref, hs2_ref, dinv_ref, out_ref):
        acc = p_ref[0, :N, :] + p_ref[1, :N, :] + hs2_ref[...]
        out_ref[...] = dinv_ref[:, :64] * acc

    return pl.pallas_call(
        body,
        out_shape=jax.ShapeDtypeStruct((N, 64), jnp.float32),
    )(part2, hs2, dinv_b)


def kernel(x, edge_index, W0, W1):
    src = edge_index[0].astype(jnp.int32)
    dst = edge_index[1].astype(jnp.int32)
    npad = EP - E
    pad = jnp.arange(npad, dtype=jnp.int32)
    src3 = jnp.concatenate([src, pad % N]).reshape(NW, CH_PER_W, CHUNK)
    dst3 = jnp.concatenate([dst, N + pad % (NPAD - N)]
                           ).reshape(NW, CH_PER_W, CHUNK)

    degp = _sc_degree(dst3)
    hs, dinv_b = _tc_layer1(x, W0, degp)
    part1 = _sc_propagate(hs, src3, dst3, 128, nbuf=2)
    hs2 = _tc_layer2(part1, hs, dinv_b, W1)
    part2 = _sc_propagate(hs2, src3, dst3, 64, nbuf=8)
    return _tc_final(part2, hs2, dinv_b)
